# Initial kernel scaffold; baseline (speedup 1.0000x reference)
import jax
import jax.numpy as jnp
from jax.experimental import pallas as pl


def kernel(edge_index, edge_type, initializations, rel_emb):
    raise NotImplementedError("write your pallas kernel here")


# speedup vs baseline: 4.8724x; 4.8724x over previous
import functools

import jax
import jax.numpy as jnp
from jax import lax
from jax.experimental import pallas as pl
from jax.experimental.pallas import tpu as pltpu
from jax.experimental.pallas import tpu_sc as plsc

N_ENTITIES = 100000
N_RELATIONS = 237
C = 50
CP = 56
E = 1600000
NW = 32
EPW = E // NW
B = 400
S = 80
NCHUNK = EPW // B


def _dist_mult_body(src_hbm, dst_hbm, typ_hbm, table_hbm, rel_hbm, out_hbm,
                    idx_s_v, idx_d_v, typ_v, rows_s_v, rows_d_v, rel_v,
                    out_v, sem):
    nc = 2
    wid = lax.axis_index("s") * nc + lax.axis_index("c")

    pltpu.sync_copy(rel_hbm, rel_v)

    iota16 = lax.iota(jnp.int32, 16)

    def chunk_body(i, carry):
        base = wid * EPW + i * B
        pltpu.sync_copy(src_hbm.at[pl.ds(base, B)], idx_s_v)
        pltpu.sync_copy(dst_hbm.at[pl.ds(base, B)], idx_d_v)
        pltpu.sync_copy(typ_hbm.at[pl.ds(base, B)], typ_v)

        copies = []
        for j in range(B // S):
            sl = pl.ds(j * S, S)
            copies.append(pltpu.async_copy(
                table_hbm.at[idx_s_v.at[sl]], rows_s_v.at[sl], sem))
            copies.append(pltpu.async_copy(
                table_hbm.at[idx_d_v.at[sl]], rows_d_v.at[sl], sem))
        for cp in copies:
            cp.wait()

        def group_body(g, carry2):
            e16 = g * 16 + iota16
            t16 = typ_v[pl.ds(g * 16, 16)]
            acc = jnp.zeros((16,), jnp.float32)
            for c in range(C):
                c16 = jnp.full((16,), c, jnp.int32)
                s = plsc.load_gather(rows_s_v, [e16, c16])
                d = plsc.load_gather(rows_d_v, [e16, c16])
                r = plsc.load_gather(rel_v, [t16, c16])
                acc = acc + s * r * d
            out_v[pl.ds(g * 16, 16)] = acc
            return carry2

        lax.fori_loop(0, B // 16, group_body, 0, unroll=False)
        pltpu.sync_copy(out_v, out_hbm.at[pl.ds(base, B)])
        return carry

    lax.fori_loop(0, NCHUNK, chunk_body, 0, unroll=False)


@jax.jit
def _dist_mult(src, dst, typ, table, rel):
    mesh = plsc.VectorSubcoreMesh(core_axis_name="c", subcore_axis_name="s")
    return pl.kernel(
        _dist_mult_body,
        out_type=jax.ShapeDtypeStruct((E,), jnp.float32),
        mesh=mesh,
        scratch_types=[
            pltpu.VMEM((B,), jnp.int32),
            pltpu.VMEM((B,), jnp.int32),
            pltpu.VMEM((B,), jnp.int32),
            pltpu.VMEM((B, CP), jnp.float32),
            pltpu.VMEM((B, CP), jnp.float32),
            pltpu.VMEM((N_RELATIONS, CP), jnp.float32),
            pltpu.VMEM((B,), jnp.float32),
            pltpu.SemaphoreType.DMA,
        ],
        compiler_params=pltpu.CompilerParams(
            needs_layout_passes=False, use_tc_tiling_on_sc=False),
    )(src, dst, typ, table, rel)


def kernel(edge_index, edge_type, initializations, rel_emb):
    table = jnp.pad(initializations, ((0, 0), (0, CP - C)))
    rel = jnp.pad(rel_emb, ((0, 0), (0, CP - C)))
    return _dist_mult(edge_index[0], edge_index[1], edge_type, table, rel)


# speedup vs baseline: 9.0726x; 1.8620x over previous
import jax
import jax.numpy as jnp
from jax import lax
from jax.experimental import pallas as pl
from jax.experimental.pallas import tpu as pltpu
from jax.experimental.pallas import tpu_sc as plsc

N_ENTITIES = 100000
N_RELATIONS = 237
C = 50
CP = 56
E = 1600000
NW = 32
EPW = E // NW
B = 400
NB = 3 * B
NCHUNK = EPW // B
NGROUP = B // 16


def _dist_mult_body(packed_hbm, table_hbm, rel_hbm, out_hbm,
                    idx_v0, idx_v1, rows_v0, rows_v1, rel_v,
                    out_v0, out_v1,
                    sem_i0, sem_i1, sem_g0, sem_g1, sem_o0, sem_o1):
    idx_v = (idx_v0, idx_v1)
    rows_v = (rows_v0, rows_v1)
    out_v = (out_v0, out_v1)
    sem_i = (sem_i0, sem_i1)
    sem_g = (sem_g0, sem_g1)
    sem_o = (sem_o0, sem_o1)

    nc = 2
    wid = lax.axis_index("s") * nc + lax.axis_index("c")

    pltpu.sync_copy(rel_hbm, rel_v)

    iota16 = lax.iota(jnp.int32, 16)

    def fire_idx(i, p):
        pltpu.async_copy(packed_hbm.at[wid * NCHUNK + i], idx_v[p], sem_i[p])

    def wait_idx(p):
        pltpu.make_async_copy(packed_hbm.at[0], idx_v[p], sem_i[p]).wait()

    def fire_gathers(p):
        pltpu.async_copy(table_hbm.at[idx_v[p].at[pl.ds(0, B)]],
                         rows_v[p].at[pl.ds(0, B)], sem_g[p])
        pltpu.async_copy(table_hbm.at[idx_v[p].at[pl.ds(B, B)]],
                         rows_v[p].at[pl.ds(B, B)], sem_g[p])

    def wait_gathers(p):
        for j in range(2):
            pltpu.make_async_copy(table_hbm.at[idx_v[p].at[pl.ds(j * B, B)]],
                                  rows_v[p].at[pl.ds(j * B, B)],
                                  sem_g[p]).wait()

    def fire_out(i, p):
        base = wid * EPW + i * B
        pltpu.async_copy(out_v[p], out_hbm.at[pl.ds(base, B)], sem_o[p])

    def wait_out(p):
        pltpu.make_async_copy(out_v[p], out_hbm.at[pl.ds(0, B)],
                              sem_o[p]).wait()

    def compute(p):
        rows = rows_v[p]
        idx = idx_v[p]
        outb = out_v[p]

        def group_body(g, carry):
            e16 = g * 16 + iota16
            t16 = idx[pl.ds(2 * B + g * 16, 16)]
            acc = jnp.zeros((16,), jnp.float32)
            for c in range(C):
                c16 = jnp.full((16,), c, jnp.int32)
                s = plsc.load_gather(rows, [e16, c16])
                d = plsc.load_gather(rows, [B + e16, c16])
                r = plsc.load_gather(rel_v, [t16, c16])
                acc = acc + s * r * d
            outb[pl.ds(g * 16, 16)] = acc
            return carry

        lax.fori_loop(0, NGROUP, group_body, 0, unroll=False)

    fire_idx(0, 0)
    wait_idx(0)
    fire_gathers(0)
    fire_idx(1, 1)

    def pair_body(k, carry):
        i0 = 2 * k
        wait_gathers(0)
        wait_idx(1)
        fire_gathers(1)

        @pl.when(k > 0)
        def _():
            wait_out(0)
        compute(0)
        fire_out(i0, 0)
        fire_idx(i0 + 2, 0)

        wait_gathers(1)
        wait_idx(0)
        fire_gathers(0)

        @pl.when(k > 0)
        def _():
            wait_out(1)
        compute(1)
        fire_out(i0 + 1, 1)

        @pl.when(i0 + 3 < NCHUNK)
        def _():
            fire_idx(i0 + 3, 1)
        return carry

    lax.fori_loop(0, (NCHUNK - 1) // 2, pair_body, 0, unroll=False)

    wait_gathers(0)
    wait_out(0)
    compute(0)
    fire_out(NCHUNK - 1, 0)
    wait_out(0)
    wait_out(1)


@jax.jit
def _dist_mult(packed, table, rel):
    mesh = plsc.VectorSubcoreMesh(core_axis_name="c", subcore_axis_name="s")
    return pl.kernel(
        _dist_mult_body,
        out_type=jax.ShapeDtypeStruct((E,), jnp.float32),
        mesh=mesh,
        scratch_types=[
            pltpu.VMEM((NB,), jnp.int32),
            pltpu.VMEM((NB,), jnp.int32),
            pltpu.VMEM((2 * B, CP), jnp.float32),
            pltpu.VMEM((2 * B, CP), jnp.float32),
            pltpu.VMEM((N_RELATIONS, CP), jnp.float32),
            pltpu.VMEM((B,), jnp.float32),
            pltpu.VMEM((B,), jnp.float32),
            pltpu.SemaphoreType.DMA,
            pltpu.SemaphoreType.DMA,
            pltpu.SemaphoreType.DMA,
            pltpu.SemaphoreType.DMA,
            pltpu.SemaphoreType.DMA,
            pltpu.SemaphoreType.DMA,
        ],
        compiler_params=pltpu.CompilerParams(
            needs_layout_passes=False, use_tc_tiling_on_sc=False),
    )(packed, table, rel)


def kernel(edge_index, edge_type, initializations, rel_emb):
    table = jnp.pad(initializations, ((0, 0), (0, CP - C)))
    rel = jnp.pad(rel_emb, ((0, 0), (0, CP - C)))
    packed = jnp.concatenate(
        [edge_index[0].reshape(NW * NCHUNK, B),
         edge_index[1].reshape(NW * NCHUNK, B),
         edge_type.reshape(NW * NCHUNK, B)], axis=1)
    return _dist_mult(packed, table, rel)


# speedup vs baseline: 10.2694x; 1.1319x over previous
import jax
import jax.numpy as jnp
from jax import lax
from jax.experimental import pallas as pl
from jax.experimental.pallas import tpu as pltpu
from jax.experimental.pallas import tpu_sc as plsc

N_ENTITIES = 100000
N_RELATIONS = 237
C = 50
CP = 56
E = 1600000
NW = 32
EPW = E // NW
B = 400
NB = 3 * B
NCHUNK = EPW // B
NGROUP = B // 16
CU = 10


def _dist_mult_body(packed_hbm, table_hbm, rel_hbm, out_hbm,
                    idx_v0, idx_v1, rows_v0, rows_v1, rel_v,
                    out_v0, out_v1,
                    sem_i0, sem_i1, sem_g0, sem_g1, sem_o0, sem_o1):
    idx_v = (idx_v0, idx_v1)
    rows_v = (rows_v0, rows_v1)
    out_v = (out_v0, out_v1)
    sem_i = (sem_i0, sem_i1)
    sem_g = (sem_g0, sem_g1)
    sem_o = (sem_o0, sem_o1)

    nc = 2
    wid = lax.axis_index("s") * nc + lax.axis_index("c")

    pltpu.sync_copy(rel_hbm, rel_v)

    iota16 = lax.iota(jnp.int32, 16)

    def fire_idx(i, p):
        pltpu.async_copy(packed_hbm.at[wid * NCHUNK + i], idx_v[p], sem_i[p])

    def wait_idx(p):
        pltpu.make_async_copy(packed_hbm.at[0], idx_v[p], sem_i[p]).wait()

    def fire_gathers(p):
        pltpu.async_copy(table_hbm.at[idx_v[p].at[pl.ds(0, B)]],
                         rows_v[p].at[pl.ds(0, B)], sem_g[p])
        pltpu.async_copy(table_hbm.at[idx_v[p].at[pl.ds(B, B)]],
                         rows_v[p].at[pl.ds(B, B)], sem_g[p])

    def wait_gathers(p):
        for j in range(2):
            pltpu.make_async_copy(table_hbm.at[idx_v[p].at[pl.ds(j * B, B)]],
                                  rows_v[p].at[pl.ds(j * B, B)],
                                  sem_g[p]).wait()

    def fire_out(i, p):
        base = wid * EPW + i * B
        pltpu.async_copy(out_v[p], out_hbm.at[pl.ds(base, B)], sem_o[p])

    def wait_out(p):
        pltpu.make_async_copy(out_v[p], out_hbm.at[pl.ds(0, B)],
                              sem_o[p]).wait()

    def compute(p):
        rows = rows_v[p]
        idx = idx_v[p]
        outb = out_v[p]

        zf = jnp.zeros((16,), jnp.float32)
        zi = jnp.zeros((16,), jnp.int32)

        def group_body(g, carry):
            e16 = g * 16 + iota16
            be16 = B + e16
            t16 = idx[pl.ds(2 * B + g * 16, 16)]

            def chan_block(cb, car):
                a0, a1, cvec = car
                for u in range(CU):
                    s = plsc.load_gather(rows, [e16, cvec])
                    d = plsc.load_gather(rows, [be16, cvec])
                    r = plsc.load_gather(rel_v, [t16, cvec])
                    if u % 2 == 0:
                        a0 = a0 + s * r * d
                    else:
                        a1 = a1 + s * r * d
                    cvec = cvec + 1
                return a0, a1, cvec

            a0, a1, _ = lax.fori_loop(0, C // CU, chan_block, (zf, zf, zi),
                                      unroll=False)
            outb[pl.ds(g * 16, 16)] = a0 + a1
            return carry

        lax.fori_loop(0, NGROUP, group_body, 0, unroll=False)

    fire_idx(0, 0)
    wait_idx(0)
    fire_gathers(0)
    fire_idx(1, 1)

    def pair_body(k, carry):
        i0 = 2 * k
        wait_gathers(0)
        wait_idx(1)
        fire_gathers(1)

        @pl.when(k > 0)
        def _():
            wait_out(0)
        compute(0)
        fire_out(i0, 0)
        fire_idx(i0 + 2, 0)

        wait_gathers(1)
        wait_idx(0)
        fire_gathers(0)

        @pl.when(k > 0)
        def _():
            wait_out(1)
        compute(1)
        fire_out(i0 + 1, 1)

        @pl.when(i0 + 3 < NCHUNK)
        def _():
            fire_idx(i0 + 3, 1)
        return carry

    lax.fori_loop(0, (NCHUNK - 1) // 2, pair_body, 0, unroll=False)

    wait_gathers(0)
    wait_out(0)
    compute(0)
    fire_out(NCHUNK - 1, 0)
    wait_out(0)
    wait_out(1)


@jax.jit
def _dist_mult(packed, table, rel):
    mesh = plsc.VectorSubcoreMesh(core_axis_name="c", subcore_axis_name="s")
    return pl.kernel(
        _dist_mult_body,
        out_type=jax.ShapeDtypeStruct((E,), jnp.float32),
        mesh=mesh,
        scratch_types=[
            pltpu.VMEM((NB,), jnp.int32),
            pltpu.VMEM((NB,), jnp.int32),
            pltpu.VMEM((2 * B, CP), jnp.float32),
            pltpu.VMEM((2 * B, CP), jnp.float32),
            pltpu.VMEM((N_RELATIONS, CP), jnp.float32),
            pltpu.VMEM((B,), jnp.float32),
            pltpu.VMEM((B,), jnp.float32),
            pltpu.SemaphoreType.DMA,
            pltpu.SemaphoreType.DMA,
            pltpu.SemaphoreType.DMA,
            pltpu.SemaphoreType.DMA,
            pltpu.SemaphoreType.DMA,
            pltpu.SemaphoreType.DMA,
        ],
        compiler_params=pltpu.CompilerParams(
            needs_layout_passes=False, use_tc_tiling_on_sc=False),
    )(packed, table, rel)


def kernel(edge_index, edge_type, initializations, rel_emb):
    table = jnp.pad(initializations, ((0, 0), (0, CP - C)))
    rel = jnp.pad(rel_emb, ((0, 0), (0, CP - C)))
    packed = jnp.concatenate(
        [edge_index[0].reshape(NW * NCHUNK, B),
         edge_index[1].reshape(NW * NCHUNK, B),
         edge_type.reshape(NW * NCHUNK, B)], axis=1)
    return _dist_mult(packed, table, rel)


# speedup vs baseline: 10.3584x; 1.0087x over previous
import jax
import jax.numpy as jnp
from jax import lax
from jax.experimental import pallas as pl
from jax.experimental.pallas import tpu as pltpu
from jax.experimental.pallas import tpu_sc as plsc

N_ENTITIES = 100000
N_RELATIONS = 237
C = 50
CP = 56
E = 1600000
NW = 32
EPW = E // NW
B = 400
NB = 3 * B
NCHUNK = EPW // B
NGROUP = B // 16
CU = 25


def _dist_mult_body(packed_hbm, table_hbm, rel_hbm, out_hbm,
                    idx_v0, idx_v1, rows_v0, rows_v1, rel_v,
                    out_v0, out_v1,
                    sem_i0, sem_i1, sem_g0, sem_g1, sem_o0, sem_o1):
    idx_v = (idx_v0, idx_v1)
    rows_v = (rows_v0, rows_v1)
    out_v = (out_v0, out_v1)
    sem_i = (sem_i0, sem_i1)
    sem_g = (sem_g0, sem_g1)
    sem_o = (sem_o0, sem_o1)

    nc = 2
    wid = lax.axis_index("s") * nc + lax.axis_index("c")

    pltpu.sync_copy(rel_hbm, rel_v)

    iota16 = lax.iota(jnp.int32, 16)

    def fire_idx(i, p):
        pltpu.async_copy(packed_hbm.at[wid * NCHUNK + i], idx_v[p], sem_i[p])

    def wait_idx(p):
        pltpu.make_async_copy(packed_hbm.at[0], idx_v[p], sem_i[p]).wait()

    def fire_gathers(p):
        pltpu.async_copy(table_hbm.at[idx_v[p].at[pl.ds(0, B)]],
                         rows_v[p].at[pl.ds(0, B)], sem_g[p])
        pltpu.async_copy(table_hbm.at[idx_v[p].at[pl.ds(B, B)]],
                         rows_v[p].at[pl.ds(B, B)], sem_g[p])

    def wait_gathers(p):
        for j in range(2):
            pltpu.make_async_copy(table_hbm.at[idx_v[p].at[pl.ds(j * B, B)]],
                                  rows_v[p].at[pl.ds(j * B, B)],
                                  sem_g[p]).wait()

    def fire_out(i, p):
        base = wid * EPW + i * B
        pltpu.async_copy(out_v[p], out_hbm.at[pl.ds(base, B)], sem_o[p])

    def wait_out(p):
        pltpu.make_async_copy(out_v[p], out_hbm.at[pl.ds(0, B)],
                              sem_o[p]).wait()

    def compute(p):
        rows = rows_v[p]
        idx = idx_v[p]
        outb = out_v[p]

        zf = jnp.zeros((16,), jnp.float32)
        zi = jnp.zeros((16,), jnp.int32)

        def group_body(g, carry):
            e16 = g * 16 + iota16
            be16 = B + e16
            t16 = idx[pl.ds(2 * B + g * 16, 16)]

            def chan_block(cb, car):
                a0, a1, cvec = car
                for u in range(CU):
                    s = plsc.load_gather(rows, [e16, cvec])
                    d = plsc.load_gather(rows, [be16, cvec])
                    r = plsc.load_gather(rel_v, [t16, cvec])
                    if u % 2 == 0:
                        a0 = a0 + s * r * d
                    else:
                        a1 = a1 + s * r * d
                    cvec = cvec + 1
                return a0, a1, cvec

            a0, a1, _ = lax.fori_loop(0, C // CU, chan_block, (zf, zf, zi),
                                      unroll=False)
            outb[pl.ds(g * 16, 16)] = a0 + a1
            return carry

        lax.fori_loop(0, NGROUP, group_body, 0, unroll=False)

    fire_idx(0, 0)
    wait_idx(0)
    fire_gathers(0)
    fire_idx(1, 1)

    def pair_body(k, carry):
        i0 = 2 * k
        wait_gathers(0)
        wait_idx(1)
        fire_gathers(1)

        @pl.when(k > 0)
        def _():
            wait_out(0)
        compute(0)
        fire_out(i0, 0)
        fire_idx(i0 + 2, 0)

        wait_gathers(1)
        wait_idx(0)
        fire_gathers(0)

        @pl.when(k > 0)
        def _():
            wait_out(1)
        compute(1)
        fire_out(i0 + 1, 1)

        @pl.when(i0 + 3 < NCHUNK)
        def _():
            fire_idx(i0 + 3, 1)
        return carry

    lax.fori_loop(0, (NCHUNK - 1) // 2, pair_body, 0, unroll=False)

    wait_gathers(0)
    wait_out(0)
    compute(0)
    fire_out(NCHUNK - 1, 0)
    wait_out(0)
    wait_out(1)


@jax.jit
def _dist_mult(packed, table, rel):
    mesh = plsc.VectorSubcoreMesh(core_axis_name="c", subcore_axis_name="s")
    return pl.kernel(
        _dist_mult_body,
        out_type=jax.ShapeDtypeStruct((E,), jnp.float32),
        mesh=mesh,
        scratch_types=[
            pltpu.VMEM((NB,), jnp.int32),
            pltpu.VMEM((NB,), jnp.int32),
            pltpu.VMEM((2 * B, CP), jnp.float32),
            pltpu.VMEM((2 * B, CP), jnp.float32),
            pltpu.VMEM((N_RELATIONS, CP), jnp.float32),
            pltpu.VMEM((B,), jnp.float32),
            pltpu.VMEM((B,), jnp.float32),
            pltpu.SemaphoreType.DMA,
            pltpu.SemaphoreType.DMA,
            pltpu.SemaphoreType.DMA,
            pltpu.SemaphoreType.DMA,
            pltpu.SemaphoreType.DMA,
            pltpu.SemaphoreType.DMA,
        ],
        compiler_params=pltpu.CompilerParams(
            needs_layout_passes=False, use_tc_tiling_on_sc=False),
    )(packed, table, rel)


def kernel(edge_index, edge_type, initializations, rel_emb):
    table = jnp.pad(initializations, ((0, 0), (0, CP - C)))
    rel = jnp.pad(rel_emb, ((0, 0), (0, CP - C)))
    packed = jnp.concatenate(
        [edge_index[0].reshape(NW * NCHUNK, B),
         edge_index[1].reshape(NW * NCHUNK, B),
         edge_type.reshape(NW * NCHUNK, B)], axis=1)
    return _dist_mult(packed, table, rel)


# speedup vs baseline: 12.2960x; 1.1870x over previous
import jax
import jax.numpy as jnp
from jax import lax
from jax.experimental import pallas as pl
from jax.experimental.pallas import tpu as pltpu
from jax.experimental.pallas import tpu_sc as plsc

N_ENTITIES = 100000
N_RELATIONS = 237
C = 50
CP = 56
E = 1600000
NW = 32
EPW = E // NW
B = 400
NB = 3 * B
NCHUNK = EPW // B
NGROUP = B // 16
CU = 10


def _dist_mult_body(packed_hbm, table_hbm, rel_hbm, out_hbm,
                    idx_v0, idx_v1, rows_v0, rows_v1, rel_v, strip_v,
                    out_v0, out_v1,
                    sem_i0, sem_i1, sem_g0, sem_g1, sem_o0, sem_o1):
    idx_v = (idx_v0, idx_v1)
    rows_v = (rows_v0, rows_v1)
    out_v = (out_v0, out_v1)
    sem_i = (sem_i0, sem_i1)
    sem_g = (sem_g0, sem_g1)
    sem_o = (sem_o0, sem_o1)

    nc = 2
    wid = lax.axis_index("s") * nc + lax.axis_index("c")

    pltpu.sync_copy(rel_hbm, rel_v)

    iota16 = lax.iota(jnp.int32, 16)

    def fire_idx(i, p):
        pltpu.async_copy(packed_hbm.at[wid * NCHUNK + i], idx_v[p], sem_i[p])

    def wait_idx(p):
        pltpu.make_async_copy(packed_hbm.at[0], idx_v[p], sem_i[p]).wait()

    def fire_gathers(p):
        pltpu.async_copy(table_hbm.at[idx_v[p].at[pl.ds(0, B)]],
                         rows_v[p].at[pl.ds(0, B)], sem_g[p])
        pltpu.async_copy(table_hbm.at[idx_v[p].at[pl.ds(B, B)]],
                         rows_v[p].at[pl.ds(B, B)], sem_g[p])

    def wait_gathers(p):
        for j in range(2):
            pltpu.make_async_copy(table_hbm.at[idx_v[p].at[pl.ds(j * B, B)]],
                                  rows_v[p].at[pl.ds(j * B, B)],
                                  sem_g[p]).wait()

    def fire_out(i, p):
        base = wid * EPW + i * B
        pltpu.async_copy(out_v[p], out_hbm.at[pl.ds(base, B)], sem_o[p])

    def wait_out(p):
        pltpu.make_async_copy(out_v[p], out_hbm.at[pl.ds(0, B)],
                              sem_o[p]).wait()

    def compute(p):
        rows = rows_v[p]
        idx = idx_v[p]
        outb = out_v[p]

        zf = jnp.zeros((16,), jnp.float32)
        le16 = jnp.right_shift(iota16, 2)
        lc16 = jnp.bitwise_and(iota16, 3)

        def group_body(g, carry):
            for j in range(4):
                e4 = g * 16 + j * 4 + le16
                be4 = B + e4
                t4 = plsc.load_gather(idx, [2 * B + e4])
                a0 = zf
                a1 = zf
                cvec = lc16
                for k in range(14):
                    s = plsc.load_gather(rows, [e4, cvec])
                    d = plsc.load_gather(rows, [be4, cvec])
                    r = plsc.load_gather(rel_v, [t4, cvec])
                    if k % 2 == 0:
                        a0 = a0 + s * r * d
                    else:
                        a1 = a1 + s * r * d
                    cvec = cvec + 4
                strip_v[pl.ds(j * 16, 16)] = a0 + a1
            a4 = iota16 * 4
            sc = zf
            for lc in range(4):
                sc = sc + plsc.load_gather(strip_v, [a4 + lc])
            outb[pl.ds(g * 16, 16)] = sc
            return carry

        lax.fori_loop(0, NGROUP, group_body, 0, unroll=False)

    fire_idx(0, 0)
    wait_idx(0)
    fire_gathers(0)
    fire_idx(1, 1)

    def pair_body(k, carry):
        i0 = 2 * k
        wait_gathers(0)
        wait_idx(1)
        fire_gathers(1)

        @pl.when(k > 0)
        def _():
            wait_out(0)
        compute(0)
        fire_out(i0, 0)
        fire_idx(i0 + 2, 0)

        wait_gathers(1)
        wait_idx(0)
        fire_gathers(0)

        @pl.when(k > 0)
        def _():
            wait_out(1)
        compute(1)
        fire_out(i0 + 1, 1)

        @pl.when(i0 + 3 < NCHUNK)
        def _():
            fire_idx(i0 + 3, 1)
        return carry

    lax.fori_loop(0, (NCHUNK - 1) // 2, pair_body, 0, unroll=False)

    wait_gathers(0)
    wait_out(0)
    compute(0)
    fire_out(NCHUNK - 1, 0)
    wait_out(0)
    wait_out(1)


@jax.jit
def _dist_mult(packed, table, rel):
    mesh = plsc.VectorSubcoreMesh(core_axis_name="c", subcore_axis_name="s")
    return pl.kernel(
        _dist_mult_body,
        out_type=jax.ShapeDtypeStruct((E,), jnp.float32),
        mesh=mesh,
        scratch_types=[
            pltpu.VMEM((NB,), jnp.int32),
            pltpu.VMEM((NB,), jnp.int32),
            pltpu.VMEM((2 * B, CP), jnp.float32),
            pltpu.VMEM((2 * B, CP), jnp.float32),
            pltpu.VMEM((N_RELATIONS, CP), jnp.float32),
            pltpu.VMEM((64,), jnp.float32),
            pltpu.VMEM((B,), jnp.float32),
            pltpu.VMEM((B,), jnp.float32),
            pltpu.SemaphoreType.DMA,
            pltpu.SemaphoreType.DMA,
            pltpu.SemaphoreType.DMA,
            pltpu.SemaphoreType.DMA,
            pltpu.SemaphoreType.DMA,
            pltpu.SemaphoreType.DMA,
        ],
        compiler_params=pltpu.CompilerParams(
            needs_layout_passes=False, use_tc_tiling_on_sc=False),
    )(packed, table, rel)


def kernel(edge_index, edge_type, initializations, rel_emb):
    table = jnp.pad(initializations, ((0, 0), (0, CP - C)))
    rel = jnp.pad(rel_emb, ((0, 0), (0, CP - C)))
    packed = jnp.concatenate(
        [edge_index[0].reshape(NW * NCHUNK, B),
         edge_index[1].reshape(NW * NCHUNK, B),
         edge_type.reshape(NW * NCHUNK, B)], axis=1)
    return _dist_mult(packed, table, rel)
